# Initial kernel scaffold; baseline (speedup 1.0000x reference)
import jax
import jax.numpy as jnp
from jax import lax
from jax.experimental import pallas as pl
from jax.experimental.pallas import tpu as pltpu
from jax.experimental.pallas import tpu_sc as plsc

N = 10000
D = 128
DH = 512
E = 160000
NC = 2
NS = 16
K = 128
CT = -(-(E // NS) // K)
EPT = CT * K
EP = NS * EPT
NP = 10240
DUMMY = N
RPT = NP // NS
BN = 1024


def _pad_edges(idx, fill):
    pad = jnp.full((EP - E,), fill, jnp.int32)
    return jnp.concatenate([idx, pad]).reshape(NS, CT, K)


def _sc1_body(x_hbm, srcA, dstA, srcR, z128, o16, z16,
              acc_out, degA_out, degR_out,
              srcA_t, dstA_t, srcR_t, rowbuf, zbuf, obuf, z16buf,
              acc_sh, dA_sh, dR_sh, sem):
    c = lax.axis_index("c")
    s = lax.axis_index("s")
    pltpu.sync_copy(srcA.at[c, s], srcA_t)
    pltpu.sync_copy(dstA.at[c, s], dstA_t)
    pltpu.sync_copy(srcR.at[c, s], srcR_t)
    pltpu.sync_copy(z128, zbuf)
    pltpu.sync_copy(o16, obuf)
    pltpu.sync_copy(z16, z16buf)
    base = s * RPT
    for k in range(RPT // K):
        pltpu.sync_copy(zbuf, acc_sh.at[pl.ds(base + k * K, K)])
        pltpu.sync_copy(z16buf, dA_sh.at[pl.ds(base + k * K, K)])
        pltpu.sync_copy(z16buf, dR_sh.at[pl.ds(base + k * K, K)])
    plsc.subcore_barrier()

    def chunk(j, carry):
        pltpu.async_copy(x_hbm.at[srcA_t.at[j]], rowbuf, sem).wait()
        pltpu.sync_copy(rowbuf, acc_sh.at[dstA_t.at[j]], add=True)
        pltpu.sync_copy(obuf, dA_sh.at[dstA_t.at[j]], add=True)
        pltpu.sync_copy(obuf, dR_sh.at[srcR_t.at[j]], add=True)
        return carry

    lax.fori_loop(0, CT, chunk, 0)
    plsc.subcore_barrier()
    pltpu.sync_copy(acc_sh.at[pl.ds(base, RPT)], acc_out.at[c, pl.ds(base, RPT)])
    pltpu.sync_copy(dA_sh.at[pl.ds(base, RPT)], degA_out.at[c, pl.ds(base, RPT)])
    pltpu.sync_copy(dR_sh.at[pl.ds(base, RPT)], degR_out.at[c, pl.ds(base, RPT)])


def _sc2_body(s2_hbm, dstR, srcR, z128,
              acc_out,
              dstR_t, srcR_t, rowbuf, zbuf, acc_sh, sem):
    c = lax.axis_index("c")
    s = lax.axis_index("s")
    pltpu.sync_copy(dstR.at[c, s], dstR_t)
    pltpu.sync_copy(srcR.at[c, s], srcR_t)
    pltpu.sync_copy(z128, zbuf)
    base = s * RPT
    for k in range(RPT // K):
        pltpu.sync_copy(zbuf, acc_sh.at[pl.ds(base + k * K, K)])
    plsc.subcore_barrier()

    def chunk(j, carry):
        pltpu.async_copy(s2_hbm.at[dstR_t.at[j]], rowbuf, sem).wait()
        pltpu.sync_copy(rowbuf, acc_sh.at[srcR_t.at[j]], add=True)
        return carry

    lax.fori_loop(0, CT, chunk, 0)
    plsc.subcore_barrier()
    pltpu.sync_copy(acc_sh.at[pl.ds(base, RPT)], acc_out.at[c, pl.ds(base, RPT)])


def _sc_mesh():
    return plsc.VectorSubcoreMesh(
        core_axis_name="c", subcore_axis_name="s", num_cores=NC, num_subcores=NS)


def _sc1(x, srcA, dstA, srcR, z128, o16, z16):
    return pl.kernel(
        _sc1_body,
        out_type=[
            jax.ShapeDtypeStruct((NC, NP, D), jnp.float32),
            jax.ShapeDtypeStruct((NC, NP, 16), jnp.float32),
            jax.ShapeDtypeStruct((NC, NP, 16), jnp.float32),
        ],
        mesh=_sc_mesh(),
        scratch_types=[
            pltpu.VMEM((CT, K), jnp.int32),
            pltpu.VMEM((CT, K), jnp.int32),
            pltpu.VMEM((CT, K), jnp.int32),
            pltpu.VMEM((K, D), jnp.float32),
            pltpu.VMEM((K, D), jnp.float32),
            pltpu.VMEM((K, 16), jnp.float32),
            pltpu.VMEM((K, 16), jnp.float32),
            pltpu.VMEM_SHARED((NP, D), jnp.float32),
            pltpu.VMEM_SHARED((NP, 16), jnp.float32),
            pltpu.VMEM_SHARED((NP, 16), jnp.float32),
            pltpu.SemaphoreType.DMA,
        ],
        name="gcn_sc_prop1",
    )(x, srcA, dstA, srcR, z128, o16, z16)


def _sc2(s2, dstR, srcR, z128):
    return pl.kernel(
        _sc2_body,
        out_type=jax.ShapeDtypeStruct((NC, NP, D), jnp.float32),
        mesh=_sc_mesh(),
        scratch_types=[
            pltpu.VMEM((CT, K), jnp.int32),
            pltpu.VMEM((CT, K), jnp.int32),
            pltpu.VMEM((K, D), jnp.float32),
            pltpu.VMEM((K, D), jnp.float32),
            pltpu.VMEM_SHARED((NP, D), jnp.float32),
            pltpu.SemaphoreType.DMA,
        ],
        name="gcn_sc_prop2",
    )(s2, dstR, srcR, z128)


def _softmax2(att_ref):
    ar = att_ref[...]
    m = jnp.maximum(ar[0, 0], ar[0, 1])
    e0 = jnp.exp(ar[0, 0] - m)
    e1 = jnp.exp(ar[0, 1] - m)
    return e0 / (e0 + e1), e1 / (e0 + e1)


def _combine(acc_ref, deg_ref, a0, a1):
    dd = deg_ref[...]
    d0 = dd[0, :, 0:1]
    d1 = dd[1, :, 0:1]
    inv0 = jnp.where(d0 > 0, a0 / jnp.where(d0 > 0, d0, 1.0), 0.0)
    inv1 = jnp.where(d1 > 0, a1 / jnp.where(d1 > 0, d1, 1.0), 0.0)
    z = acc_ref[0] * inv0 + acc_ref[1] * inv1
    cmask = jnp.where(d0 > 0, a0, 0.0) + jnp.where(d1 > 0, a1, 0.0)
    return z, cmask


def _tc1_body(acc_ref, deg_ref, att_ref, w1_ref, b1_ref, w2_ref, b2_ref, out_ref):
    a0, a1 = _softmax2(att_ref)
    z, cmask = _combine(acc_ref, deg_ref, a0, a1)
    h = jnp.dot(z, w1_ref[...], preferred_element_type=jnp.float32)
    h = h + cmask * b1_ref[...]
    h = jnp.where(h > 0, h, 0.2 * h)
    s2 = jnp.dot(h, w2_ref[...], preferred_element_type=jnp.float32)
    out_ref[...] = s2 + b2_ref[...]


def _tc1(acc, deg, att, W1, b1, W2, b2):
    return pl.pallas_call(
        _tc1_body,
        grid=(NP // BN,),
        in_specs=[
            pl.BlockSpec((NC, BN, D), lambda i: (0, i, 0)),
            pl.BlockSpec((NC, BN, 16), lambda i: (0, i, 0)),
            pl.BlockSpec((1, 2), lambda i: (0, 0)),
            pl.BlockSpec((D, DH), lambda i: (0, 0)),
            pl.BlockSpec((1, DH), lambda i: (0, 0)),
            pl.BlockSpec((DH, D), lambda i: (0, 0)),
            pl.BlockSpec((1, D), lambda i: (0, 0)),
        ],
        out_specs=pl.BlockSpec((BN, D), lambda i: (i, 0)),
        out_shape=jax.ShapeDtypeStruct((NP, D), jnp.float32),
        name="gcn_tc_dense",
    )(acc, deg, att, W1, b1, W2, b2)


def _tc2_body(acc_ref, deg_ref, att_ref, out_ref):
    a0, a1 = _softmax2(att_ref)
    o, _ = _combine(acc_ref, deg_ref, a0, a1)
    nrm = jnp.maximum(jnp.sqrt(jnp.sum(o * o, axis=1, keepdims=True)), 1e-12)
    out_ref[...] = o / nrm


def _tc2(acc, deg, att):
    return pl.pallas_call(
        _tc2_body,
        grid=(NP // BN,),
        in_specs=[
            pl.BlockSpec((NC, BN, D), lambda i: (0, i, 0)),
            pl.BlockSpec((NC, BN, 16), lambda i: (0, i, 0)),
            pl.BlockSpec((1, 2), lambda i: (0, 0)),
        ],
        out_specs=pl.BlockSpec((BN, D), lambda i: (i, 0)),
        out_shape=jax.ShapeDtypeStruct((NP, D), jnp.float32),
        name="gcn_tc_norm",
    )(acc, deg, att)


def kernel(x, W1, b1, W2, b2, a_att, r_att, src0, dst0, src1, dst1):
    srcA = jnp.stack([_pad_edges(src0, 0), _pad_edges(src1, 0)])
    dstA = jnp.stack([_pad_edges(dst0, DUMMY), _pad_edges(dst1, DUMMY)])
    srcR = jnp.stack([_pad_edges(src0, DUMMY), _pad_edges(src1, DUMMY)])
    dstR = jnp.stack([_pad_edges(dst0, 0), _pad_edges(dst1, 0)])
    z128 = jnp.zeros((K, D), jnp.float32)
    o16 = jnp.ones((K, 16), jnp.float32)
    z16 = jnp.zeros((K, 16), jnp.float32)

    accA, degA, degR = _sc1(x, srcA, dstA, srcR, z128, o16, z16)
    s2 = _tc1(accA, degA, a_att.reshape(1, 2), W1, b1.reshape(1, DH),
              W2, b2.reshape(1, D))
    accR = _sc2(s2, dstR, srcR, z128)
    o = _tc2(accR, degR, r_att.reshape(1, 2))
    return o[:N]


# speedup vs baseline: 6.2801x; 6.2801x over previous
import jax
import jax.numpy as jnp
from jax import lax
from jax.experimental import pallas as pl
from jax.experimental.pallas import tpu as pltpu
from jax.experimental.pallas import tpu_sc as plsc

N = 10000
D = 128
DH = 512
E = 160000
NC = 2
NS = 16
L = 16
K = 128
CT = -(-(E // NS) // K)
EP = NS * CT * K
NP = 10112
DUMMY = N
RPT = NP // NS
DG = 80
BN = 1264


def _pad_edges(idx):
    pad = jnp.full((EP - E,), DUMMY, jnp.int32)
    return jnp.concatenate([idx, pad]).reshape(NS, CT, K)


def _sc1_body(x_hbm, eidx, zeros_h, iota_h,
              acc_out, degA_out, degR_out,
              ebuf, rowbuf, degA_t, degR_t, iota_t,
              acc_sh, dA_sh, dR_sh, sem):
    c = lax.axis_index("c")
    s = lax.axis_index("s")
    pltpu.sync_copy(iota_h, iota_t)
    pltpu.sync_copy(zeros_h.at[pl.ds(0, DG)], degA_t)
    pltpu.sync_copy(zeros_h.at[pl.ds(0, DG)], degR_t)
    base = s * RPT
    pltpu.sync_copy(zeros_h.at[pl.ds(0, RPT)], acc_sh.at[pl.ds(base, RPT)])

    @pl.when(s < DG // 8)
    def _():
        pltpu.sync_copy(zeros_h.at[pl.ds(0, 8)], dA_sh.at[pl.ds(s * 8, 8)])
        pltpu.sync_copy(zeros_h.at[pl.ds(0, 8)], dR_sh.at[pl.ds(s * 8, 8)])

    plsc.subcore_barrier()

    def chunk(j, carry):
        pltpu.sync_copy(eidx.at[c, s, j], ebuf)
        pltpu.async_copy(x_hbm.at[ebuf.at[0]], rowbuf, sem).wait()
        pltpu.sync_copy(rowbuf, acc_sh.at[ebuf.at[1]], add=True)
        for v in range(K // L):
            dv = ebuf[1, pl.ds(v * L, L)]
            cnt, last = plsc.scan_count(dv)
            plsc.addupdate_scatter(
                degA_t,
                [lax.shift_right_logical(dv, 7), jnp.bitwise_and(dv, 127)],
                cnt.astype(jnp.float32), mask=last)
            sv = ebuf[0, pl.ds(v * L, L)]
            cnt2, last2 = plsc.scan_count(sv)
            plsc.addupdate_scatter(
                degR_t,
                [lax.shift_right_logical(sv, 7), jnp.bitwise_and(sv, 127)],
                cnt2.astype(jnp.float32), mask=last2)
        return carry

    lax.fori_loop(0, CT, chunk, 0)
    pltpu.sync_copy(degA_t, dA_sh.at[iota_t.at[0]], add=True)
    pltpu.sync_copy(degR_t, dR_sh.at[iota_t.at[0]], add=True)
    plsc.subcore_barrier()
    pltpu.sync_copy(acc_sh.at[pl.ds(base, RPT)], acc_out.at[c, pl.ds(base, RPT)])

    @pl.when(s < DG // 8)
    def _():
        pltpu.sync_copy(dA_sh.at[pl.ds(s * 8, 8)], degA_out.at[c, pl.ds(s * 8, 8)])
        pltpu.sync_copy(dR_sh.at[pl.ds(s * 8, 8)], degR_out.at[c, pl.ds(s * 8, 8)])


def _sc2_body(s2_hbm, eidx, zeros_h,
              acc_out,
              ebuf, rowbuf, acc_sh, sem):
    c = lax.axis_index("c")
    s = lax.axis_index("s")
    base = s * RPT
    pltpu.sync_copy(zeros_h.at[pl.ds(0, RPT)], acc_sh.at[pl.ds(base, RPT)])
    plsc.subcore_barrier()

    def chunk(j, carry):
        pltpu.sync_copy(eidx.at[c, s, j], ebuf)
        pltpu.async_copy(s2_hbm.at[ebuf.at[1]], rowbuf, sem).wait()
        pltpu.sync_copy(rowbuf, acc_sh.at[ebuf.at[0]], add=True)
        return carry

    lax.fori_loop(0, CT, chunk, 0)
    plsc.subcore_barrier()
    pltpu.sync_copy(acc_sh.at[pl.ds(base, RPT)], acc_out.at[c, pl.ds(base, RPT)])


def _sc_mesh():
    return plsc.VectorSubcoreMesh(
        core_axis_name="c", subcore_axis_name="s", num_cores=NC, num_subcores=NS)


def _sc1(x, eidx, zeros_h, iota_h):
    return pl.kernel(
        _sc1_body,
        out_type=[
            jax.ShapeDtypeStruct((NC, NP, D), jnp.float32),
            jax.ShapeDtypeStruct((NC, DG, 128), jnp.float32),
            jax.ShapeDtypeStruct((NC, DG, 128), jnp.float32),
        ],
        mesh=_sc_mesh(),
        scratch_types=[
            pltpu.VMEM((2, K), jnp.int32),
            pltpu.VMEM((K, D), jnp.float32),
            pltpu.VMEM((DG, 128), jnp.float32),
            pltpu.VMEM((DG, 128), jnp.float32),
            pltpu.VMEM((1, DG), jnp.int32),
            pltpu.VMEM_SHARED((NP, D), jnp.float32),
            pltpu.VMEM_SHARED((DG, 128), jnp.float32),
            pltpu.VMEM_SHARED((DG, 128), jnp.float32),
            pltpu.SemaphoreType.DMA,
        ],
        compiler_params=pltpu.CompilerParams(needs_layout_passes=False),
        name="gcn_sc_prop1",
    )(x, eidx, zeros_h, iota_h)


def _sc2(s2, eidx, zeros_h):
    return pl.kernel(
        _sc2_body,
        out_type=jax.ShapeDtypeStruct((NC, NP, D), jnp.float32),
        mesh=_sc_mesh(),
        scratch_types=[
            pltpu.VMEM((2, K), jnp.int32),
            pltpu.VMEM((K, D), jnp.float32),
            pltpu.VMEM_SHARED((NP, D), jnp.float32),
            pltpu.SemaphoreType.DMA,
        ],
        compiler_params=pltpu.CompilerParams(needs_layout_passes=False),
        name="gcn_sc_prop2",
    )(s2, eidx, zeros_h)


def _softmax2(att_ref):
    ar = att_ref[...]
    m = jnp.maximum(ar[0, 0], ar[0, 1])
    e0 = jnp.exp(ar[0, 0] - m)
    e1 = jnp.exp(ar[0, 1] - m)
    return e0 / (e0 + e1), e1 / (e0 + e1)


def _combine(acc_ref, deg_ref, a0, a1):
    dd = deg_ref[...]
    d0 = dd[0]
    d1 = dd[1]
    inv0 = jnp.where(d0 > 0, a0 / jnp.where(d0 > 0, d0, 1.0), 0.0)
    inv1 = jnp.where(d1 > 0, a1 / jnp.where(d1 > 0, d1, 1.0), 0.0)
    z = acc_ref[0] * inv0 + acc_ref[1] * inv1
    cmask = jnp.where(d0 > 0, a0, 0.0) + jnp.where(d1 > 0, a1, 0.0)
    return z, cmask


def _tc1_body(acc_ref, deg_ref, att_ref, w1_ref, b1_ref, w2_ref, b2_ref, out_ref):
    a0, a1 = _softmax2(att_ref)
    z, cmask = _combine(acc_ref, deg_ref, a0, a1)
    h = jnp.dot(z, w1_ref[...], preferred_element_type=jnp.float32)
    h = h + cmask * b1_ref[...]
    h = jnp.where(h > 0, h, 0.2 * h)
    s2 = jnp.dot(h, w2_ref[...], preferred_element_type=jnp.float32)
    out_ref[...] = s2 + b2_ref[...]


def _tc1(acc, deg, att, W1, b1, W2, b2):
    return pl.pallas_call(
        _tc1_body,
        grid=(NP // BN,),
        in_specs=[
            pl.BlockSpec((NC, BN, D), lambda i: (0, i, 0)),
            pl.BlockSpec((NC, BN, 1), lambda i: (0, i, 0)),
            pl.BlockSpec((1, 2), lambda i: (0, 0)),
            pl.BlockSpec((D, DH), lambda i: (0, 0)),
            pl.BlockSpec((1, DH), lambda i: (0, 0)),
            pl.BlockSpec((DH, D), lambda i: (0, 0)),
            pl.BlockSpec((1, D), lambda i: (0, 0)),
        ],
        out_specs=pl.BlockSpec((BN, D), lambda i: (i, 0)),
        out_shape=jax.ShapeDtypeStruct((NP, D), jnp.float32),
        name="gcn_tc_dense",
    )(acc, deg, att, W1, b1, W2, b2)


def _tc2_body(acc_ref, deg_ref, att_ref, out_ref):
    a0, a1 = _softmax2(att_ref)
    o, _ = _combine(acc_ref, deg_ref, a0, a1)
    nrm = jnp.maximum(jnp.sqrt(jnp.sum(o * o, axis=1, keepdims=True)), 1e-12)
    out_ref[...] = o / nrm


def _tc2(acc, deg, att):
    return pl.pallas_call(
        _tc2_body,
        grid=(NP // BN,),
        in_specs=[
            pl.BlockSpec((NC, BN, D), lambda i: (0, i, 0)),
            pl.BlockSpec((NC, BN, 1), lambda i: (0, i, 0)),
            pl.BlockSpec((1, 2), lambda i: (0, 0)),
        ],
        out_specs=pl.BlockSpec((BN, D), lambda i: (i, 0)),
        out_shape=jax.ShapeDtypeStruct((NP, D), jnp.float32),
        name="gcn_tc_norm",
    )(acc, deg, att)


def _deg_to_col(deg):
    return deg.reshape(NC, DG * 128)[:, :NP].reshape(NC, NP, 1)


def kernel(x, W1, b1, W2, b2, a_att, r_att, src0, dst0, src1, dst1):
    eidx = jnp.stack([
        jnp.stack([_pad_edges(src0), _pad_edges(dst0)], axis=2),
        jnp.stack([_pad_edges(src1), _pad_edges(dst1)], axis=2),
    ])
    x_pad = jnp.concatenate([x, jnp.zeros((NP - N, D), jnp.float32)])
    zeros_h = jnp.zeros((RPT, D), jnp.float32)
    iota_h = jnp.arange(DG, dtype=jnp.int32).reshape(1, DG)

    accA, degA, degR = _sc1(x_pad, eidx, zeros_h, iota_h)
    s2 = _tc1(accA, _deg_to_col(degA), a_att.reshape(1, 2), W1,
              b1.reshape(1, DH), W2, b2.reshape(1, D))
    accR = _sc2(s2, eidx, zeros_h)
    o = _tc2(accR, _deg_to_col(degR), r_att.reshape(1, 2))
    return o[:N]


# speedup vs baseline: 6.9226x; 1.1023x over previous
import jax
import jax.numpy as jnp
from jax import lax
from jax.experimental import pallas as pl
from jax.experimental.pallas import tpu as pltpu
from jax.experimental.pallas import tpu_sc as plsc

N = 10000
D = 128
DH = 512
E = 160000
NC = 2
NS = 16
L = 16
K = 128
CT = -(-(E // NS) // K)
EP = NS * CT * K
NP = 10112
DUMMY = N
RPT = NP // NS
DG = 80
BN = 1264


def _pad_edges(idx):
    pad = jnp.full((EP - E,), DUMMY, jnp.int32)
    return jnp.concatenate([idx, pad]).reshape(NS, CT, K)


def _sc1_body(x_hbm, eidx, zeros_h, iota_h,
              acc_out, degA_out,
              ebuf, rowbuf, degA_t, iota_t,
              acc_sh, dA_sh, sem_i, sem_g, sem_s):
    c = lax.axis_index("c")
    s = lax.axis_index("s")
    pltpu.sync_copy(iota_h, iota_t)
    pltpu.sync_copy(zeros_h.at[pl.ds(0, DG)], degA_t)
    base = s * RPT
    pltpu.sync_copy(zeros_h.at[pl.ds(0, RPT)], acc_sh.at[pl.ds(base, RPT)])

    @pl.when(s < DG // 8)
    def _():
        pltpu.sync_copy(zeros_h.at[pl.ds(0, 8)], dA_sh.at[pl.ds(s * 8, 8)])

    plsc.subcore_barrier()

    pltpu.async_copy(eidx.at[c, s, 0], ebuf.at[0], sem_i)

    def chunk(j, carry):
        b = lax.rem(j, 2)
        pltpu.make_async_copy(eidx.at[c, s, j], ebuf.at[b], sem_i).wait()

        @pl.when(j >= 1)
        def _():
            pltpu.make_async_copy(
                rowbuf.at[1 - b], acc_sh.at[ebuf.at[1 - b, 1]], sem_s).wait()

        @pl.when(j + 1 < CT)
        def _():
            pltpu.async_copy(eidx.at[c, s, j + 1], ebuf.at[1 - b], sem_i)

        pltpu.async_copy(x_hbm.at[ebuf.at[b, 0]], rowbuf.at[b], sem_g).wait()
        pltpu.async_copy(rowbuf.at[b], acc_sh.at[ebuf.at[b, 1]], sem_s,
                         add=True)
        for v in range(K // L):
            dv = ebuf[b, 1, pl.ds(v * L, L)]
            cnt, last = plsc.scan_count(dv)
            plsc.addupdate_scatter(
                degA_t,
                [lax.shift_right_logical(dv, 7), jnp.bitwise_and(dv, 127)],
                cnt.astype(jnp.float32), mask=last)
        return carry

    lax.fori_loop(0, CT, chunk, 0)
    lb = (CT - 1) % 2
    pltpu.make_async_copy(
        rowbuf.at[lb], acc_sh.at[ebuf.at[lb, 1]], sem_s).wait()
    pltpu.sync_copy(degA_t, dA_sh.at[iota_t.at[0]], add=True)
    plsc.subcore_barrier()
    pltpu.sync_copy(acc_sh.at[pl.ds(base, RPT)], acc_out.at[c, pl.ds(base, RPT)])

    @pl.when(s < DG // 8)
    def _():
        pltpu.sync_copy(dA_sh.at[pl.ds(s * 8, 8)], degA_out.at[c, pl.ds(s * 8, 8)])


def _sc2_body(s2_hbm, eidx, zeros_h, iota_h,
              acc_out, degR_out,
              ebuf, rowbuf, degR_t, iota_t, acc_sh, dR_sh,
              sem_i, sem_g, sem_s):
    c = lax.axis_index("c")
    s = lax.axis_index("s")
    pltpu.sync_copy(iota_h, iota_t)
    pltpu.sync_copy(zeros_h.at[pl.ds(0, DG)], degR_t)
    base = s * RPT
    pltpu.sync_copy(zeros_h.at[pl.ds(0, RPT)], acc_sh.at[pl.ds(base, RPT)])

    @pl.when(s < DG // 8)
    def _():
        pltpu.sync_copy(zeros_h.at[pl.ds(0, 8)], dR_sh.at[pl.ds(s * 8, 8)])

    plsc.subcore_barrier()

    pltpu.async_copy(eidx.at[c, s, 0], ebuf.at[0], sem_i)

    def chunk(j, carry):
        b = lax.rem(j, 2)
        pltpu.make_async_copy(eidx.at[c, s, j], ebuf.at[b], sem_i).wait()

        @pl.when(j >= 1)
        def _():
            pltpu.make_async_copy(
                rowbuf.at[1 - b], acc_sh.at[ebuf.at[1 - b, 0]], sem_s).wait()

        @pl.when(j + 1 < CT)
        def _():
            pltpu.async_copy(eidx.at[c, s, j + 1], ebuf.at[1 - b], sem_i)

        pltpu.async_copy(s2_hbm.at[ebuf.at[b, 1]], rowbuf.at[b], sem_g).wait()
        pltpu.async_copy(rowbuf.at[b], acc_sh.at[ebuf.at[b, 0]], sem_s,
                         add=True)
        for v in range(K // L):
            sv = ebuf[b, 0, pl.ds(v * L, L)]
            cnt, last = plsc.scan_count(sv)
            plsc.addupdate_scatter(
                degR_t,
                [lax.shift_right_logical(sv, 7), jnp.bitwise_and(sv, 127)],
                cnt.astype(jnp.float32), mask=last)
        return carry

    lax.fori_loop(0, CT, chunk, 0)
    lb = (CT - 1) % 2
    pltpu.make_async_copy(
        rowbuf.at[lb], acc_sh.at[ebuf.at[lb, 0]], sem_s).wait()
    pltpu.sync_copy(degR_t, dR_sh.at[iota_t.at[0]], add=True)
    plsc.subcore_barrier()
    pltpu.sync_copy(acc_sh.at[pl.ds(base, RPT)], acc_out.at[c, pl.ds(base, RPT)])

    @pl.when(s < DG // 8)
    def _():
        pltpu.sync_copy(dR_sh.at[pl.ds(s * 8, 8)], degR_out.at[c, pl.ds(s * 8, 8)])


def _sc_mesh():
    return plsc.VectorSubcoreMesh(
        core_axis_name="c", subcore_axis_name="s", num_cores=NC, num_subcores=NS)


def _sc1(x, eidx, zeros_h, iota_h):
    return pl.kernel(
        _sc1_body,
        out_type=[
            jax.ShapeDtypeStruct((NC, NP, D), jnp.float32),
            jax.ShapeDtypeStruct((NC, DG, 128), jnp.float32),
        ],
        mesh=_sc_mesh(),
        scratch_types=[
            pltpu.VMEM((2, 2, K), jnp.int32),
            pltpu.VMEM((2, K, D), jnp.float32),
            pltpu.VMEM((DG, 128), jnp.float32),
            pltpu.VMEM((1, DG), jnp.int32),
            pltpu.VMEM_SHARED((NP, D), jnp.float32),
            pltpu.VMEM_SHARED((DG, 128), jnp.float32),
            pltpu.SemaphoreType.DMA,
            pltpu.SemaphoreType.DMA,
            pltpu.SemaphoreType.DMA,
        ],
        compiler_params=pltpu.CompilerParams(needs_layout_passes=False),
        name="gcn_sc_prop1",
    )(x, eidx, zeros_h, iota_h)


def _sc2(s2, eidx, zeros_h, iota_h):
    return pl.kernel(
        _sc2_body,
        out_type=[
            jax.ShapeDtypeStruct((NC, NP, D), jnp.float32),
            jax.ShapeDtypeStruct((NC, DG, 128), jnp.float32),
        ],
        mesh=_sc_mesh(),
        scratch_types=[
            pltpu.VMEM((2, 2, K), jnp.int32),
            pltpu.VMEM((2, K, D), jnp.float32),
            pltpu.VMEM((DG, 128), jnp.float32),
            pltpu.VMEM((1, DG), jnp.int32),
            pltpu.VMEM_SHARED((NP, D), jnp.float32),
            pltpu.VMEM_SHARED((DG, 128), jnp.float32),
            pltpu.SemaphoreType.DMA,
            pltpu.SemaphoreType.DMA,
            pltpu.SemaphoreType.DMA,
        ],
        compiler_params=pltpu.CompilerParams(needs_layout_passes=False),
        name="gcn_sc_prop2",
    )(s2, eidx, zeros_h, iota_h)


def _softmax2(att_ref):
    ar = att_ref[...]
    m = jnp.maximum(ar[0, 0], ar[0, 1])
    e0 = jnp.exp(ar[0, 0] - m)
    e1 = jnp.exp(ar[0, 1] - m)
    return e0 / (e0 + e1), e1 / (e0 + e1)


def _combine(acc_ref, deg_ref, a0, a1):
    dd = deg_ref[...]
    d0 = dd[0]
    d1 = dd[1]
    inv0 = jnp.where(d0 > 0, a0 / jnp.where(d0 > 0, d0, 1.0), 0.0)
    inv1 = jnp.where(d1 > 0, a1 / jnp.where(d1 > 0, d1, 1.0), 0.0)
    z = acc_ref[0] * inv0 + acc_ref[1] * inv1
    cmask = jnp.where(d0 > 0, a0, 0.0) + jnp.where(d1 > 0, a1, 0.0)
    return z, cmask


def _tc1_body(acc_ref, deg_ref, att_ref, w1_ref, b1_ref, w2_ref, b2_ref, out_ref):
    a0, a1 = _softmax2(att_ref)
    z, cmask = _combine(acc_ref, deg_ref, a0, a1)
    h = jnp.dot(z, w1_ref[...], preferred_element_type=jnp.float32)
    h = h + cmask * b1_ref[...]
    h = jnp.where(h > 0, h, 0.2 * h)
    s2 = jnp.dot(h, w2_ref[...], preferred_element_type=jnp.float32)
    out_ref[...] = s2 + b2_ref[...]


def _tc1(acc, deg, att, W1, b1, W2, b2):
    return pl.pallas_call(
        _tc1_body,
        grid=(NP // BN,),
        in_specs=[
            pl.BlockSpec((NC, BN, D), lambda i: (0, i, 0)),
            pl.BlockSpec((NC, BN, 1), lambda i: (0, i, 0)),
            pl.BlockSpec((1, 2), lambda i: (0, 0)),
            pl.BlockSpec((D, DH), lambda i: (0, 0)),
            pl.BlockSpec((1, DH), lambda i: (0, 0)),
            pl.BlockSpec((DH, D), lambda i: (0, 0)),
            pl.BlockSpec((1, D), lambda i: (0, 0)),
        ],
        out_specs=pl.BlockSpec((BN, D), lambda i: (i, 0)),
        out_shape=jax.ShapeDtypeStruct((NP, D), jnp.float32),
        name="gcn_tc_dense",
    )(acc, deg, att, W1, b1, W2, b2)


def _tc2_body(acc_ref, deg_ref, att_ref, out_ref):
    a0, a1 = _softmax2(att_ref)
    o, _ = _combine(acc_ref, deg_ref, a0, a1)
    nrm = jnp.maximum(jnp.sqrt(jnp.sum(o * o, axis=1, keepdims=True)), 1e-12)
    out_ref[...] = o / nrm


def _tc2(acc, deg, att):
    return pl.pallas_call(
        _tc2_body,
        grid=(NP // BN,),
        in_specs=[
            pl.BlockSpec((NC, BN, D), lambda i: (0, i, 0)),
            pl.BlockSpec((NC, BN, 1), lambda i: (0, i, 0)),
            pl.BlockSpec((1, 2), lambda i: (0, 0)),
        ],
        out_specs=pl.BlockSpec((BN, D), lambda i: (i, 0)),
        out_shape=jax.ShapeDtypeStruct((NP, D), jnp.float32),
        name="gcn_tc_norm",
    )(acc, deg, att)


def _deg_to_col(deg):
    return deg.reshape(NC, DG * 128)[:, :NP].reshape(NC, NP, 1)


def kernel(x, W1, b1, W2, b2, a_att, r_att, src0, dst0, src1, dst1):
    eidx = jnp.stack([
        jnp.stack([_pad_edges(src0), _pad_edges(dst0)], axis=2),
        jnp.stack([_pad_edges(src1), _pad_edges(dst1)], axis=2),
    ])
    x_pad = jnp.concatenate([x, jnp.zeros((NP - N, D), jnp.float32)])
    zeros_h = jnp.zeros((RPT, D), jnp.float32)
    iota_h = jnp.arange(DG, dtype=jnp.int32).reshape(1, DG)

    accA, degA = _sc1(x_pad, eidx, zeros_h, iota_h)
    s2 = _tc1(accA, _deg_to_col(degA), a_att.reshape(1, 2), W1,
              b1.reshape(1, DH), W2, b2.reshape(1, D))
    accR, degR = _sc2(s2, eidx, zeros_h, iota_h)
    o = _tc2(accR, _deg_to_col(degR), r_att.reshape(1, 2))
    return o[:N]


# speedup vs baseline: 10.8578x; 1.5685x over previous
import jax
import jax.numpy as jnp
from jax import lax
from jax.experimental import pallas as pl
from jax.experimental.pallas import tpu as pltpu
from jax.experimental.pallas import tpu_sc as plsc

N = 10000
D = 128
DH = 512
E = 160000
NC = 2
NS = 16
L = 16
K = 64
R = 4
CT = -(-(E // NS) // K)
EP = NS * CT * K
NP = 10112
DUMMY = N
RPT = NP // NS
DG = 80
BN = 1264


def _pad_edges(idx):
    pad = jnp.full((EP - E,), DUMMY, jnp.int32)
    return jnp.concatenate([idx, pad]).reshape(NS, CT, K)


def _prop_loop(hbm_src, eidx, c, s, ebuf, rowbuf, acc_sh,
               sem_i, sem_g, sem_s, g_row, s_row, hist_fn):
    for p in range(3):
        pltpu.async_copy(eidx.at[c, s, p], ebuf.at[p % R], sem_i)
    for p in range(2):
        pltpu.make_async_copy(eidx.at[c, s, p], ebuf.at[p % R], sem_i).wait()
        pltpu.async_copy(
            hbm_src.at[ebuf.at[p % R, g_row]], rowbuf.at[p % R], sem_g)

    def chunk(j, carry):
        b = lax.rem(j, R)
        pltpu.make_async_copy(
            hbm_src.at[ebuf.at[b, g_row]], rowbuf.at[b], sem_g).wait()
        pltpu.async_copy(rowbuf.at[b], acc_sh.at[ebuf.at[b, s_row]], sem_s,
                         add=True)
        hist_fn(b)

        @pl.when(j >= 1)
        def _():
            bm1 = lax.rem(j + R - 1, R)
            pltpu.make_async_copy(
                rowbuf.at[bm1], acc_sh.at[ebuf.at[bm1, s_row]], sem_s).wait()

        @pl.when(j + 3 < CT)
        def _():
            b3 = lax.rem(j + 3, R)
            pltpu.async_copy(eidx.at[c, s, j + 3], ebuf.at[b3], sem_i)

        @pl.when(j + 2 < CT)
        def _():
            b2 = lax.rem(j + 2, R)
            pltpu.make_async_copy(
                eidx.at[c, s, j + 2], ebuf.at[b2], sem_i).wait()
            pltpu.async_copy(
                hbm_src.at[ebuf.at[b2, g_row]], rowbuf.at[b2], sem_g)

        return carry

    lax.fori_loop(0, CT, chunk, 0)
    lb = (CT - 1) % R
    pltpu.make_async_copy(
        rowbuf.at[lb], acc_sh.at[ebuf.at[lb, s_row]], sem_s).wait()


def _sc1_body(x_hbm, eidx, zeros_h, iota_h,
              acc_out, degA_out,
              ebuf, rowbuf, degA_t, iota_t,
              acc_sh, dA_sh, sem_i, sem_g, sem_s):
    c = lax.axis_index("c")
    s = lax.axis_index("s")
    pltpu.sync_copy(iota_h, iota_t)
    pltpu.sync_copy(zeros_h.at[pl.ds(0, DG)], degA_t)
    base = s * RPT
    pltpu.sync_copy(zeros_h.at[pl.ds(0, RPT)], acc_sh.at[pl.ds(base, RPT)])

    @pl.when(s < DG // 8)
    def _():
        pltpu.sync_copy(zeros_h.at[pl.ds(0, 8)], dA_sh.at[pl.ds(s * 8, 8)])

    plsc.subcore_barrier()

    def hist(b):
        for v in range(K // L):
            dv = ebuf[b, 1, pl.ds(v * L, L)]
            cnt, last = plsc.scan_count(dv)
            plsc.addupdate_scatter(
                degA_t,
                [lax.shift_right_logical(dv, 7), jnp.bitwise_and(dv, 127)],
                cnt.astype(jnp.float32), mask=last)

    _prop_loop(x_hbm, eidx, c, s, ebuf, rowbuf, acc_sh,
               sem_i, sem_g, sem_s, 0, 1, hist)
    pltpu.sync_copy(degA_t, dA_sh.at[iota_t.at[0]], add=True)
    plsc.subcore_barrier()
    pltpu.sync_copy(acc_sh.at[pl.ds(base, RPT)], acc_out.at[c, pl.ds(base, RPT)])

    @pl.when(s < DG // 8)
    def _():
        pltpu.sync_copy(dA_sh.at[pl.ds(s * 8, 8)], degA_out.at[c, pl.ds(s * 8, 8)])


def _sc2_body(s2_hbm, eidx, zeros_h, iota_h,
              acc_out, degR_out,
              ebuf, rowbuf, degR_t, iota_t, acc_sh, dR_sh,
              sem_i, sem_g, sem_s):
    c = lax.axis_index("c")
    s = lax.axis_index("s")
    pltpu.sync_copy(iota_h, iota_t)
    pltpu.sync_copy(zeros_h.at[pl.ds(0, DG)], degR_t)
    base = s * RPT
    pltpu.sync_copy(zeros_h.at[pl.ds(0, RPT)], acc_sh.at[pl.ds(base, RPT)])

    @pl.when(s < DG // 8)
    def _():
        pltpu.sync_copy(zeros_h.at[pl.ds(0, 8)], dR_sh.at[pl.ds(s * 8, 8)])

    plsc.subcore_barrier()

    def hist(b):
        for v in range(K // L):
            sv = ebuf[b, 0, pl.ds(v * L, L)]
            cnt, last = plsc.scan_count(sv)
            plsc.addupdate_scatter(
                degR_t,
                [lax.shift_right_logical(sv, 7), jnp.bitwise_and(sv, 127)],
                cnt.astype(jnp.float32), mask=last)

    _prop_loop(s2_hbm, eidx, c, s, ebuf, rowbuf, acc_sh,
               sem_i, sem_g, sem_s, 1, 0, hist)
    pltpu.sync_copy(degR_t, dR_sh.at[iota_t.at[0]], add=True)
    plsc.subcore_barrier()
    pltpu.sync_copy(acc_sh.at[pl.ds(base, RPT)], acc_out.at[c, pl.ds(base, RPT)])

    @pl.when(s < DG // 8)
    def _():
        pltpu.sync_copy(dR_sh.at[pl.ds(s * 8, 8)], degR_out.at[c, pl.ds(s * 8, 8)])


def _sc_mesh():
    return plsc.VectorSubcoreMesh(
        core_axis_name="c", subcore_axis_name="s", num_cores=NC, num_subcores=NS)


def _sc1(x, eidx, zeros_h, iota_h):
    return pl.kernel(
        _sc1_body,
        out_type=[
            jax.ShapeDtypeStruct((NC, NP, D), jnp.float32),
            jax.ShapeDtypeStruct((NC, DG, 128), jnp.float32),
        ],
        mesh=_sc_mesh(),
        scratch_types=[
            pltpu.VMEM((R, 2, K), jnp.int32),
            pltpu.VMEM((R, K, D), jnp.float32),
            pltpu.VMEM((DG, 128), jnp.float32),
            pltpu.VMEM((1, DG), jnp.int32),
            pltpu.VMEM_SHARED((NP, D), jnp.float32),
            pltpu.VMEM_SHARED((DG, 128), jnp.float32),
            pltpu.SemaphoreType.DMA,
            pltpu.SemaphoreType.DMA,
            pltpu.SemaphoreType.DMA,
        ],
        compiler_params=pltpu.CompilerParams(needs_layout_passes=False),
        name="gcn_sc_prop1",
    )(x, eidx, zeros_h, iota_h)


def _sc2(s2, eidx, zeros_h, iota_h):
    return pl.kernel(
        _sc2_body,
        out_type=[
            jax.ShapeDtypeStruct((NC, NP, D), jnp.float32),
            jax.ShapeDtypeStruct((NC, DG, 128), jnp.float32),
        ],
        mesh=_sc_mesh(),
        scratch_types=[
            pltpu.VMEM((R, 2, K), jnp.int32),
            pltpu.VMEM((R, K, D), jnp.float32),
            pltpu.VMEM((DG, 128), jnp.float32),
            pltpu.VMEM((1, DG), jnp.int32),
            pltpu.VMEM_SHARED((NP, D), jnp.float32),
            pltpu.VMEM_SHARED((DG, 128), jnp.float32),
            pltpu.SemaphoreType.DMA,
            pltpu.SemaphoreType.DMA,
            pltpu.SemaphoreType.DMA,
        ],
        compiler_params=pltpu.CompilerParams(needs_layout_passes=False),
        name="gcn_sc_prop2",
    )(s2, eidx, zeros_h, iota_h)


def _softmax2(att_ref):
    ar = att_ref[...]
    m = jnp.maximum(ar[0, 0], ar[0, 1])
    e0 = jnp.exp(ar[0, 0] - m)
    e1 = jnp.exp(ar[0, 1] - m)
    return e0 / (e0 + e1), e1 / (e0 + e1)


def _combine(acc_ref, deg_ref, a0, a1):
    dd = deg_ref[...]
    d0 = dd[0]
    d1 = dd[1]
    inv0 = jnp.where(d0 > 0, a0 / jnp.where(d0 > 0, d0, 1.0), 0.0)
    inv1 = jnp.where(d1 > 0, a1 / jnp.where(d1 > 0, d1, 1.0), 0.0)
    z = acc_ref[0] * inv0 + acc_ref[1] * inv1
    cmask = jnp.where(d0 > 0, a0, 0.0) + jnp.where(d1 > 0, a1, 0.0)
    return z, cmask


def _tc1_body(acc_ref, deg_ref, att_ref, w1_ref, b1_ref, w2_ref, b2_ref, out_ref):
    a0, a1 = _softmax2(att_ref)
    z, cmask = _combine(acc_ref, deg_ref, a0, a1)
    h = jnp.dot(z, w1_ref[...], preferred_element_type=jnp.float32)
    h = h + cmask * b1_ref[...]
    h = jnp.where(h > 0, h, 0.2 * h)
    s2 = jnp.dot(h, w2_ref[...], preferred_element_type=jnp.float32)
    out_ref[...] = s2 + b2_ref[...]


def _tc1(acc, deg, att, W1, b1, W2, b2):
    return pl.pallas_call(
        _tc1_body,
        grid=(NP // BN,),
        in_specs=[
            pl.BlockSpec((NC, BN, D), lambda i: (0, i, 0)),
            pl.BlockSpec((NC, BN, 1), lambda i: (0, i, 0)),
            pl.BlockSpec((1, 2), lambda i: (0, 0)),
            pl.BlockSpec((D, DH), lambda i: (0, 0)),
            pl.BlockSpec((1, DH), lambda i: (0, 0)),
            pl.BlockSpec((DH, D), lambda i: (0, 0)),
            pl.BlockSpec((1, D), lambda i: (0, 0)),
        ],
        out_specs=pl.BlockSpec((BN, D), lambda i: (i, 0)),
        out_shape=jax.ShapeDtypeStruct((NP, D), jnp.float32),
        name="gcn_tc_dense",
    )(acc, deg, att, W1, b1, W2, b2)


def _tc2_body(acc_ref, deg_ref, att_ref, out_ref):
    a0, a1 = _softmax2(att_ref)
    o, _ = _combine(acc_ref, deg_ref, a0, a1)
    nrm = jnp.maximum(jnp.sqrt(jnp.sum(o * o, axis=1, keepdims=True)), 1e-12)
    out_ref[...] = o / nrm


def _tc2(acc, deg, att):
    return pl.pallas_call(
        _tc2_body,
        grid=(NP // BN,),
        in_specs=[
            pl.BlockSpec((NC, BN, D), lambda i: (0, i, 0)),
            pl.BlockSpec((NC, BN, 1), lambda i: (0, i, 0)),
            pl.BlockSpec((1, 2), lambda i: (0, 0)),
        ],
        out_specs=pl.BlockSpec((BN, D), lambda i: (i, 0)),
        out_shape=jax.ShapeDtypeStruct((NP, D), jnp.float32),
        name="gcn_tc_norm",
    )(acc, deg, att)


def _deg_to_col(deg):
    return deg.reshape(NC, DG * 128)[:, :NP].reshape(NC, NP, 1)


def kernel(x, W1, b1, W2, b2, a_att, r_att, src0, dst0, src1, dst1):
    eidx = jnp.stack([
        jnp.stack([_pad_edges(src0), _pad_edges(dst0)], axis=2),
        jnp.stack([_pad_edges(src1), _pad_edges(dst1)], axis=2),
    ])
    x_pad = jnp.concatenate([x, jnp.zeros((NP - N, D), jnp.float32)])
    zeros_h = jnp.zeros((RPT, D), jnp.float32)
    iota_h = jnp.arange(DG, dtype=jnp.int32).reshape(1, DG)

    accA, degA = _sc1(x_pad, eidx, zeros_h, iota_h)
    s2 = _tc1(accA, _deg_to_col(degA), a_att.reshape(1, 2), W1,
              b1.reshape(1, DH), W2, b2.reshape(1, D))
    accR, degR = _sc2(s2, eidx, zeros_h, iota_h)
    o = _tc2(accR, _deg_to_col(degR), r_att.reshape(1, 2))
    return o[:N]


# speedup vs baseline: 10.9671x; 1.0101x over previous
import jax
import jax.numpy as jnp
from jax import lax
from jax.experimental import pallas as pl
from jax.experimental.pallas import tpu as pltpu
from jax.experimental.pallas import tpu_sc as plsc

N = 10000
D = 128
DH = 512
E = 160000
NC = 2
NS = 16
L = 16
K = 64
R = 4
CT = -(-(E // NS) // K)
EP = NS * CT * K
NP = 10112
DUMMY = N
RPT = NP // NS
DG = 80
BN = 1264


def _pad_edges(idx):
    pad = jnp.full((EP - E,), DUMMY, jnp.int32)
    return jnp.concatenate([idx, pad]).reshape(NS, CT, K)


def _prop_loop(hbm_src, eidx, c, s, ebuf, rowbuf, acc_sh,
               sem_i, sem_g, sem_s, g_row, s_row, hist_fn):
    for p in range(3):
        pltpu.async_copy(eidx.at[c, s, p], ebuf.at[p % (2 * R)], sem_i)
    for p in range(2):
        pltpu.make_async_copy(
            eidx.at[c, s, p], ebuf.at[p % (2 * R)], sem_i).wait()
        pltpu.async_copy(
            hbm_src.at[ebuf.at[p % (2 * R), g_row]], rowbuf.at[p % R], sem_g)

    def chunk(j, carry):
        b = lax.rem(j, R)
        e = lax.rem(j, 2 * R)
        pltpu.make_async_copy(
            hbm_src.at[ebuf.at[e, g_row]], rowbuf.at[b], sem_g).wait()
        pltpu.async_copy(rowbuf.at[b], acc_sh.at[ebuf.at[e, s_row]], sem_s,
                         add=True)

        @pl.when(j >= 2)
        def _():
            bm2 = lax.rem(j + R - 2, R)
            em2 = lax.rem(j + 2 * R - 2, 2 * R)
            pltpu.make_async_copy(
                rowbuf.at[bm2], acc_sh.at[ebuf.at[em2, s_row]], sem_s).wait()

        @pl.when(j + 3 < CT)
        def _():
            e3 = lax.rem(j + 3, 2 * R)
            pltpu.async_copy(eidx.at[c, s, j + 3], ebuf.at[e3], sem_i)

        @pl.when(j + 2 < CT)
        def _():
            b2 = lax.rem(j + 2, R)
            e2 = lax.rem(j + 2, 2 * R)
            pltpu.make_async_copy(
                eidx.at[c, s, j + 2], ebuf.at[e2], sem_i).wait()
            pltpu.async_copy(
                hbm_src.at[ebuf.at[e2, g_row]], rowbuf.at[b2], sem_g)

        hist_fn(e)
        return carry

    lax.fori_loop(0, CT, chunk, 0)
    for p in (CT - 2, CT - 1):
        pltpu.make_async_copy(
            rowbuf.at[p % R], acc_sh.at[ebuf.at[p % (2 * R), s_row]],
            sem_s).wait()


def _sc1_body(x_hbm, eidx, zeros_h, iota_h,
              acc_out, degA_out,
              ebuf, rowbuf, degA_t, iota_t,
              acc_sh, dA_sh, sem_i, sem_g, sem_s):
    c = lax.axis_index("c")
    s = lax.axis_index("s")
    pltpu.sync_copy(iota_h, iota_t)
    pltpu.sync_copy(zeros_h.at[pl.ds(0, DG)], degA_t)
    base = s * RPT
    pltpu.sync_copy(zeros_h.at[pl.ds(0, RPT)], acc_sh.at[pl.ds(base, RPT)])

    @pl.when(s < DG // 8)
    def _():
        pltpu.sync_copy(zeros_h.at[pl.ds(0, 8)], dA_sh.at[pl.ds(s * 8, 8)])

    plsc.subcore_barrier()

    def hist(b):
        for v in range(K // L):
            dv = ebuf[b, 1, pl.ds(v * L, L)]
            cnt, last = plsc.scan_count(dv)
            plsc.addupdate_scatter(
                degA_t,
                [lax.shift_right_logical(dv, 7), jnp.bitwise_and(dv, 127)],
                cnt.astype(jnp.float32), mask=last)

    _prop_loop(x_hbm, eidx, c, s, ebuf, rowbuf, acc_sh,
               sem_i, sem_g, sem_s, 0, 1, hist)
    pltpu.sync_copy(degA_t, dA_sh.at[iota_t.at[0]], add=True)
    plsc.subcore_barrier()
    pltpu.sync_copy(acc_sh.at[pl.ds(base, RPT)], acc_out.at[c, pl.ds(base, RPT)])

    @pl.when(s < DG // 8)
    def _():
        pltpu.sync_copy(dA_sh.at[pl.ds(s * 8, 8)], degA_out.at[c, pl.ds(s * 8, 8)])


def _sc2_body(s2_hbm, eidx, zeros_h, iota_h,
              acc_out, degR_out,
              ebuf, rowbuf, degR_t, iota_t, acc_sh, dR_sh,
              sem_i, sem_g, sem_s):
    c = lax.axis_index("c")
    s = lax.axis_index("s")
    pltpu.sync_copy(iota_h, iota_t)
    pltpu.sync_copy(zeros_h.at[pl.ds(0, DG)], degR_t)
    base = s * RPT
    pltpu.sync_copy(zeros_h.at[pl.ds(0, RPT)], acc_sh.at[pl.ds(base, RPT)])

    @pl.when(s < DG // 8)
    def _():
        pltpu.sync_copy(zeros_h.at[pl.ds(0, 8)], dR_sh.at[pl.ds(s * 8, 8)])

    plsc.subcore_barrier()

    def hist(b):
        for v in range(K // L):
            sv = ebuf[b, 0, pl.ds(v * L, L)]
            cnt, last = plsc.scan_count(sv)
            plsc.addupdate_scatter(
                degR_t,
                [lax.shift_right_logical(sv, 7), jnp.bitwise_and(sv, 127)],
                cnt.astype(jnp.float32), mask=last)

    _prop_loop(s2_hbm, eidx, c, s, ebuf, rowbuf, acc_sh,
               sem_i, sem_g, sem_s, 1, 0, hist)
    pltpu.sync_copy(degR_t, dR_sh.at[iota_t.at[0]], add=True)
    plsc.subcore_barrier()
    pltpu.sync_copy(acc_sh.at[pl.ds(base, RPT)], acc_out.at[c, pl.ds(base, RPT)])

    @pl.when(s < DG // 8)
    def _():
        pltpu.sync_copy(dR_sh.at[pl.ds(s * 8, 8)], degR_out.at[c, pl.ds(s * 8, 8)])


def _sc_mesh():
    return plsc.VectorSubcoreMesh(
        core_axis_name="c", subcore_axis_name="s", num_cores=NC, num_subcores=NS)


def _sc1(x, eidx, zeros_h, iota_h):
    return pl.kernel(
        _sc1_body,
        out_type=[
            jax.ShapeDtypeStruct((NC, NP, D), jnp.float32),
            jax.ShapeDtypeStruct((NC, DG, 128), jnp.float32),
        ],
        mesh=_sc_mesh(),
        scratch_types=[
            pltpu.VMEM((2 * R, 2, K), jnp.int32),
            pltpu.VMEM((R, K, D), jnp.float32),
            pltpu.VMEM((DG, 128), jnp.float32),
            pltpu.VMEM((1, DG), jnp.int32),
            pltpu.VMEM_SHARED((NP, D), jnp.float32),
            pltpu.VMEM_SHARED((DG, 128), jnp.float32),
            pltpu.SemaphoreType.DMA,
            pltpu.SemaphoreType.DMA,
            pltpu.SemaphoreType.DMA,
        ],
        compiler_params=pltpu.CompilerParams(needs_layout_passes=False),
        name="gcn_sc_prop1",
    )(x, eidx, zeros_h, iota_h)


def _sc2(s2, eidx, zeros_h, iota_h):
    return pl.kernel(
        _sc2_body,
        out_type=[
            jax.ShapeDtypeStruct((NC, NP, D), jnp.float32),
            jax.ShapeDtypeStruct((NC, DG, 128), jnp.float32),
        ],
        mesh=_sc_mesh(),
        scratch_types=[
            pltpu.VMEM((2 * R, 2, K), jnp.int32),
            pltpu.VMEM((R, K, D), jnp.float32),
            pltpu.VMEM((DG, 128), jnp.float32),
            pltpu.VMEM((1, DG), jnp.int32),
            pltpu.VMEM_SHARED((NP, D), jnp.float32),
            pltpu.VMEM_SHARED((DG, 128), jnp.float32),
            pltpu.SemaphoreType.DMA,
            pltpu.SemaphoreType.DMA,
            pltpu.SemaphoreType.DMA,
        ],
        compiler_params=pltpu.CompilerParams(needs_layout_passes=False),
        name="gcn_sc_prop2",
    )(s2, eidx, zeros_h, iota_h)


def _softmax2(att_ref):
    ar = att_ref[...]
    m = jnp.maximum(ar[0, 0], ar[0, 1])
    e0 = jnp.exp(ar[0, 0] - m)
    e1 = jnp.exp(ar[0, 1] - m)
    return e0 / (e0 + e1), e1 / (e0 + e1)


def _combine(acc_ref, deg_ref, a0, a1):
    dd = deg_ref[...]
    d0 = dd[0]
    d1 = dd[1]
    inv0 = jnp.where(d0 > 0, a0 / jnp.where(d0 > 0, d0, 1.0), 0.0)
    inv1 = jnp.where(d1 > 0, a1 / jnp.where(d1 > 0, d1, 1.0), 0.0)
    z = acc_ref[0] * inv0 + acc_ref[1] * inv1
    cmask = jnp.where(d0 > 0, a0, 0.0) + jnp.where(d1 > 0, a1, 0.0)
    return z, cmask


def _tc1_body(acc_ref, deg_ref, att_ref, w1_ref, b1_ref, w2_ref, b2_ref, out_ref):
    a0, a1 = _softmax2(att_ref)
    z, cmask = _combine(acc_ref, deg_ref, a0, a1)
    h = jnp.dot(z, w1_ref[...], preferred_element_type=jnp.float32)
    h = h + cmask * b1_ref[...]
    h = jnp.where(h > 0, h, 0.2 * h)
    s2 = jnp.dot(h, w2_ref[...], preferred_element_type=jnp.float32)
    out_ref[...] = s2 + b2_ref[...]


def _tc1(acc, deg, att, W1, b1, W2, b2):
    return pl.pallas_call(
        _tc1_body,
        grid=(NP // BN,),
        in_specs=[
            pl.BlockSpec((NC, BN, D), lambda i: (0, i, 0)),
            pl.BlockSpec((NC, BN, 1), lambda i: (0, i, 0)),
            pl.BlockSpec((1, 2), lambda i: (0, 0)),
            pl.BlockSpec((D, DH), lambda i: (0, 0)),
            pl.BlockSpec((1, DH), lambda i: (0, 0)),
            pl.BlockSpec((DH, D), lambda i: (0, 0)),
            pl.BlockSpec((1, D), lambda i: (0, 0)),
        ],
        out_specs=pl.BlockSpec((BN, D), lambda i: (i, 0)),
        out_shape=jax.ShapeDtypeStruct((NP, D), jnp.float32),
        name="gcn_tc_dense",
    )(acc, deg, att, W1, b1, W2, b2)


def _tc2_body(acc_ref, deg_ref, att_ref, out_ref):
    a0, a1 = _softmax2(att_ref)
    o, _ = _combine(acc_ref, deg_ref, a0, a1)
    nrm = jnp.maximum(jnp.sqrt(jnp.sum(o * o, axis=1, keepdims=True)), 1e-12)
    out_ref[...] = o / nrm


def _tc2(acc, deg, att):
    return pl.pallas_call(
        _tc2_body,
        grid=(NP // BN,),
        in_specs=[
            pl.BlockSpec((NC, BN, D), lambda i: (0, i, 0)),
            pl.BlockSpec((NC, BN, 1), lambda i: (0, i, 0)),
            pl.BlockSpec((1, 2), lambda i: (0, 0)),
        ],
        out_specs=pl.BlockSpec((BN, D), lambda i: (i, 0)),
        out_shape=jax.ShapeDtypeStruct((NP, D), jnp.float32),
        name="gcn_tc_norm",
    )(acc, deg, att)


def _deg_to_col(deg):
    return deg.reshape(NC, DG * 128)[:, :NP].reshape(NC, NP, 1)


def kernel(x, W1, b1, W2, b2, a_att, r_att, src0, dst0, src1, dst1):
    eidx = jnp.stack([
        jnp.stack([_pad_edges(src0), _pad_edges(dst0)], axis=2),
        jnp.stack([_pad_edges(src1), _pad_edges(dst1)], axis=2),
    ])
    x_pad = jnp.concatenate([x, jnp.zeros((NP - N, D), jnp.float32)])
    zeros_h = jnp.zeros((RPT, D), jnp.float32)
    iota_h = jnp.arange(DG, dtype=jnp.int32).reshape(1, DG)

    accA, degA = _sc1(x_pad, eidx, zeros_h, iota_h)
    s2 = _tc1(accA, _deg_to_col(degA), a_att.reshape(1, 2), W1,
              b1.reshape(1, DH), W2, b2.reshape(1, D))
    accR, degR = _sc2(s2, eidx, zeros_h, iota_h)
    o = _tc2(accR, _deg_to_col(degR), r_att.reshape(1, 2))
    return o[:N]


# speedup vs baseline: 12.0156x; 1.0956x over previous
import jax
import jax.numpy as jnp
from jax import lax
from jax.experimental import pallas as pl
from jax.experimental.pallas import tpu as pltpu
from jax.experimental.pallas import tpu_sc as plsc

N = 10000
D = 128
DH = 512
E = 160000
NC = 2
NS = 16
L = 16
K = 48
G = 3
R = G + 2
ER = 2 * R
CT = -(-(E // NS) // K)
EP = NS * CT * K
NP = 10112
DUMMY = N
RPT = NP // NS
DG = 80
BN = 2000


def _pad_edges(idx):
    pad = jnp.full((EP - E,), DUMMY, jnp.int32)
    return jnp.concatenate([idx, pad]).reshape(NS, CT, K)


def _prop_loop(hbm_src, eidx, c, s, ebuf, rowbuf, acc_sh,
               sem_i, sem_g, sem_s, g_row, s_row, hist_fn):
    for p in range(G + 1):
        pltpu.async_copy(eidx.at[c, s, p], ebuf.at[p % ER], sem_i)
    for p in range(G):
        pltpu.make_async_copy(
            eidx.at[c, s, p], ebuf.at[p % ER], sem_i).wait()
        pltpu.async_copy(
            hbm_src.at[ebuf.at[p % ER, g_row]], rowbuf.at[p % R], sem_g)

    def chunk(j, carry):
        b = lax.rem(j, R)
        e = lax.rem(j, ER)
        pltpu.make_async_copy(
            hbm_src.at[ebuf.at[e, g_row]], rowbuf.at[b], sem_g).wait()
        pltpu.async_copy(rowbuf.at[b], acc_sh.at[ebuf.at[e, s_row]], sem_s,
                         add=True)

        @pl.when(j >= 2)
        def _():
            bm2 = lax.rem(j + R - 2, R)
            em2 = lax.rem(j + ER - 2, ER)
            pltpu.make_async_copy(
                rowbuf.at[bm2], acc_sh.at[ebuf.at[em2, s_row]], sem_s).wait()

        @pl.when(j + G + 1 < CT)
        def _():
            e3 = lax.rem(j + G + 1, ER)
            pltpu.async_copy(eidx.at[c, s, j + G + 1], ebuf.at[e3], sem_i)

        @pl.when(j + G < CT)
        def _():
            b2 = lax.rem(j + G, R)
            e2 = lax.rem(j + G, ER)
            pltpu.make_async_copy(
                eidx.at[c, s, j + G], ebuf.at[e2], sem_i).wait()
            pltpu.async_copy(
                hbm_src.at[ebuf.at[e2, g_row]], rowbuf.at[b2], sem_g)

        hist_fn(e)
        return carry

    lax.fori_loop(0, CT, chunk, 0)
    for p in (CT - 2, CT - 1):
        pltpu.make_async_copy(
            rowbuf.at[p % R], acc_sh.at[ebuf.at[p % ER, s_row]],
            sem_s).wait()


def _sc1_body(x_hbm, eidx, zeros_h, iota_h,
              acc_out, degA_out,
              ebuf, rowbuf, degA_t, iota_t,
              acc_sh, dA_sh, sem_i, sem_g, sem_s):
    c = lax.axis_index("c")
    s = lax.axis_index("s")
    pltpu.sync_copy(iota_h, iota_t)
    pltpu.sync_copy(zeros_h.at[pl.ds(0, DG)], degA_t)
    base = s * RPT
    pltpu.sync_copy(zeros_h.at[pl.ds(0, RPT)], acc_sh.at[pl.ds(base, RPT)])

    @pl.when(s < DG // 8)
    def _():
        pltpu.sync_copy(zeros_h.at[pl.ds(0, 8)], dA_sh.at[pl.ds(s * 8, 8)])

    plsc.subcore_barrier()

    def hist(b):
        for v in range(K // L):
            dv = ebuf[b, 1, pl.ds(v * L, L)]
            cnt, last = plsc.scan_count(dv)
            plsc.addupdate_scatter(
                degA_t,
                [lax.shift_right_logical(dv, 7), jnp.bitwise_and(dv, 127)],
                cnt.astype(jnp.float32), mask=last)

    _prop_loop(x_hbm, eidx, c, s, ebuf, rowbuf, acc_sh,
               sem_i, sem_g, sem_s, 0, 1, hist)
    pltpu.sync_copy(degA_t, dA_sh.at[iota_t.at[0]], add=True)
    plsc.subcore_barrier()
    pltpu.sync_copy(acc_sh.at[pl.ds(base, RPT)], acc_out.at[c, pl.ds(base, RPT)])

    @pl.when(s < DG // 8)
    def _():
        pltpu.sync_copy(dA_sh.at[pl.ds(s * 8, 8)], degA_out.at[c, pl.ds(s * 8, 8)])


def _sc2_body(s2_hbm, eidx, zeros_h, iota_h,
              acc_out, degR_out,
              ebuf, rowbuf, degR_t, iota_t, acc_sh, dR_sh,
              sem_i, sem_g, sem_s):
    c = lax.axis_index("c")
    s = lax.axis_index("s")
    pltpu.sync_copy(iota_h, iota_t)
    pltpu.sync_copy(zeros_h.at[pl.ds(0, DG)], degR_t)
    base = s * RPT
    pltpu.sync_copy(zeros_h.at[pl.ds(0, RPT)], acc_sh.at[pl.ds(base, RPT)])

    @pl.when(s < DG // 8)
    def _():
        pltpu.sync_copy(zeros_h.at[pl.ds(0, 8)], dR_sh.at[pl.ds(s * 8, 8)])

    plsc.subcore_barrier()

    def hist(b):
        for v in range(K // L):
            sv = ebuf[b, 0, pl.ds(v * L, L)]
            cnt, last = plsc.scan_count(sv)
            plsc.addupdate_scatter(
                degR_t,
                [lax.shift_right_logical(sv, 7), jnp.bitwise_and(sv, 127)],
                cnt.astype(jnp.float32), mask=last)

    _prop_loop(s2_hbm, eidx, c, s, ebuf, rowbuf, acc_sh,
               sem_i, sem_g, sem_s, 1, 0, hist)
    pltpu.sync_copy(degR_t, dR_sh.at[iota_t.at[0]], add=True)
    plsc.subcore_barrier()
    pltpu.sync_copy(acc_sh.at[pl.ds(base, RPT)], acc_out.at[c, pl.ds(base, RPT)])

    @pl.when(s < DG // 8)
    def _():
        pltpu.sync_copy(dR_sh.at[pl.ds(s * 8, 8)], degR_out.at[c, pl.ds(s * 8, 8)])


def _sc_mesh():
    return plsc.VectorSubcoreMesh(
        core_axis_name="c", subcore_axis_name="s", num_cores=NC, num_subcores=NS)


def _sc1(x, eidx, zeros_h, iota_h):
    return pl.kernel(
        _sc1_body,
        out_type=[
            jax.ShapeDtypeStruct((NC, NP, D), jnp.float32),
            jax.ShapeDtypeStruct((NC, DG, 128), jnp.float32),
        ],
        mesh=_sc_mesh(),
        scratch_types=[
            pltpu.VMEM((2 * R, 2, K), jnp.int32),
            pltpu.VMEM((R, K, D), jnp.float32),
            pltpu.VMEM((DG, 128), jnp.float32),
            pltpu.VMEM((1, DG), jnp.int32),
            pltpu.VMEM_SHARED((NP, D), jnp.float32),
            pltpu.VMEM_SHARED((DG, 128), jnp.float32),
            pltpu.SemaphoreType.DMA,
            pltpu.SemaphoreType.DMA,
            pltpu.SemaphoreType.DMA,
        ],
        compiler_params=pltpu.CompilerParams(needs_layout_passes=False),
        name="gcn_sc_prop1",
    )(x, eidx, zeros_h, iota_h)


def _sc2(s2, eidx, zeros_h, iota_h):
    return pl.kernel(
        _sc2_body,
        out_type=[
            jax.ShapeDtypeStruct((NC, NP, D), jnp.float32),
            jax.ShapeDtypeStruct((NC, DG, 128), jnp.float32),
        ],
        mesh=_sc_mesh(),
        scratch_types=[
            pltpu.VMEM((2 * R, 2, K), jnp.int32),
            pltpu.VMEM((R, K, D), jnp.float32),
            pltpu.VMEM((DG, 128), jnp.float32),
            pltpu.VMEM((1, DG), jnp.int32),
            pltpu.VMEM_SHARED((NP, D), jnp.float32),
            pltpu.VMEM_SHARED((DG, 128), jnp.float32),
            pltpu.SemaphoreType.DMA,
            pltpu.SemaphoreType.DMA,
            pltpu.SemaphoreType.DMA,
        ],
        compiler_params=pltpu.CompilerParams(needs_layout_passes=False),
        name="gcn_sc_prop2",
    )(s2, eidx, zeros_h, iota_h)


def _softmax2(att_ref):
    ar = att_ref[...]
    m = jnp.maximum(ar[0, 0], ar[0, 1])
    e0 = jnp.exp(ar[0, 0] - m)
    e1 = jnp.exp(ar[0, 1] - m)
    return e0 / (e0 + e1), e1 / (e0 + e1)


def _combine(acc_ref, deg_ref, a0, a1):
    dd = deg_ref[...]
    d0 = dd[0]
    d1 = dd[1]
    inv0 = jnp.where(d0 > 0, a0 / jnp.where(d0 > 0, d0, 1.0), 0.0)
    inv1 = jnp.where(d1 > 0, a1 / jnp.where(d1 > 0, d1, 1.0), 0.0)
    z = acc_ref[0] * inv0 + acc_ref[1] * inv1
    cmask = jnp.where(d0 > 0, a0, 0.0) + jnp.where(d1 > 0, a1, 0.0)
    return z, cmask


def _tc1_body(acc_ref, deg_ref, att_ref, w1_ref, b1_ref, w2_ref, b2_ref, out_ref):
    a0, a1 = _softmax2(att_ref)
    z, cmask = _combine(acc_ref, deg_ref, a0, a1)
    h = jnp.dot(z, w1_ref[...], preferred_element_type=jnp.float32)
    h = h + cmask * b1_ref[...]
    h = jnp.where(h > 0, h, 0.2 * h)
    s2 = jnp.dot(h, w2_ref[...], preferred_element_type=jnp.float32)
    out_ref[...] = s2 + b2_ref[...]


def _tc1(acc, deg, att, W1, b1, W2, b2):
    return pl.pallas_call(
        _tc1_body,
        grid=(N // BN,),
        in_specs=[
            pl.BlockSpec((NC, BN, D), lambda i: (0, i, 0)),
            pl.BlockSpec((NC, BN, 1), lambda i: (0, i, 0)),
            pl.BlockSpec((1, 2), lambda i: (0, 0)),
            pl.BlockSpec((D, DH), lambda i: (0, 0)),
            pl.BlockSpec((1, DH), lambda i: (0, 0)),
            pl.BlockSpec((DH, D), lambda i: (0, 0)),
            pl.BlockSpec((1, D), lambda i: (0, 0)),
        ],
        out_specs=pl.BlockSpec((BN, D), lambda i: (i, 0)),
        out_shape=jax.ShapeDtypeStruct((NP, D), jnp.float32),
        name="gcn_tc_dense",
    )(acc, deg, att, W1, b1, W2, b2)


def _tc2_body(acc_ref, deg_ref, att_ref, out_ref):
    a0, a1 = _softmax2(att_ref)
    o, _ = _combine(acc_ref, deg_ref, a0, a1)
    nrm = jnp.maximum(jnp.sqrt(jnp.sum(o * o, axis=1, keepdims=True)), 1e-12)
    out_ref[...] = o / nrm


def _tc2(acc, deg, att):
    return pl.pallas_call(
        _tc2_body,
        grid=(N // BN,),
        in_specs=[
            pl.BlockSpec((NC, BN, D), lambda i: (0, i, 0)),
            pl.BlockSpec((NC, BN, 1), lambda i: (0, i, 0)),
            pl.BlockSpec((1, 2), lambda i: (0, 0)),
        ],
        out_specs=pl.BlockSpec((BN, D), lambda i: (i, 0)),
        out_shape=jax.ShapeDtypeStruct((N, D), jnp.float32),
        name="gcn_tc_norm",
    )(acc, deg, att)


def _deg_to_col(deg):
    return deg.reshape(NC, DG * 128, 1)


def kernel(x, W1, b1, W2, b2, a_att, r_att, src0, dst0, src1, dst1):
    eidx = jnp.stack([
        jnp.stack([_pad_edges(src0), _pad_edges(dst0)], axis=2),
        jnp.stack([_pad_edges(src1), _pad_edges(dst1)], axis=2),
    ])
    x_pad = jnp.concatenate([x, jnp.zeros((NP - N, D), jnp.float32)])
    zeros_h = jnp.zeros((RPT, D), jnp.float32)
    iota_h = jnp.arange(DG, dtype=jnp.int32).reshape(1, DG)

    accA, degA = _sc1(x_pad, eidx, zeros_h, iota_h)
    s2 = _tc1(accA, _deg_to_col(degA), a_att.reshape(1, 2), W1,
              b1.reshape(1, DH), W2, b2.reshape(1, D))
    accR, degR = _sc2(s2, eidx, zeros_h, iota_h)
    return _tc2(accR, _deg_to_col(degR), r_att.reshape(1, 2))


# speedup vs baseline: 12.4802x; 1.0387x over previous
import jax
import jax.numpy as jnp
from jax import lax
from jax.experimental import pallas as pl
from jax.experimental.pallas import tpu as pltpu
from jax.experimental.pallas import tpu_sc as plsc

N = 10000
D = 128
DH = 512
E = 160000
NC = 2
NS = 16
L = 16
K = 48
G = 4
R = G + 2
ER = G + 3
CT = -(-(E // NS) // K)
EP = NS * CT * K
NP = 10112
DUMMY = N
RPT = NP // NS
DG = 80
BN = 2000


def _pad_edges(idx):
    pad = jnp.full((EP - E,), DUMMY, jnp.int32)
    return jnp.concatenate([idx, pad]).reshape(NS, CT, K)


def _prop_loop(hbm_src, eidx, c, s, ebuf, rowbuf, acc_sh,
               sem_i, sem_g, sem_s, g_row, s_row, hist_fn):
    for p in range(G + 1):
        pltpu.async_copy(eidx.at[c, s, p], ebuf.at[p % ER], sem_i)
    for p in range(G):
        pltpu.make_async_copy(
            eidx.at[c, s, p], ebuf.at[p % ER], sem_i).wait()
        pltpu.async_copy(
            hbm_src.at[ebuf.at[p % ER, g_row]], rowbuf.at[p % R], sem_g)

    def chunk(j, carry):
        b = lax.rem(j, R)
        e = lax.rem(j, ER)
        pltpu.make_async_copy(
            hbm_src.at[ebuf.at[e, g_row]], rowbuf.at[b], sem_g).wait()
        pltpu.async_copy(rowbuf.at[b], acc_sh.at[ebuf.at[e, s_row]], sem_s,
                         add=True)

        @pl.when(j >= 2)
        def _():
            bm2 = lax.rem(j + R - 2, R)
            em2 = lax.rem(j + ER - 2, ER)
            pltpu.make_async_copy(
                rowbuf.at[bm2], acc_sh.at[ebuf.at[em2, s_row]], sem_s).wait()

        @pl.when(j + G + 1 < CT)
        def _():
            e3 = lax.rem(j + G + 1, ER)
            pltpu.async_copy(eidx.at[c, s, j + G + 1], ebuf.at[e3], sem_i)

        @pl.when(j + G < CT)
        def _():
            b2 = lax.rem(j + G, R)
            e2 = lax.rem(j + G, ER)
            pltpu.make_async_copy(
                eidx.at[c, s, j + G], ebuf.at[e2], sem_i).wait()
            pltpu.async_copy(
                hbm_src.at[ebuf.at[e2, g_row]], rowbuf.at[b2], sem_g)

        hist_fn(e)
        return carry

    lax.fori_loop(0, CT, chunk, 0)
    for p in (CT - 2, CT - 1):
        pltpu.make_async_copy(
            rowbuf.at[p % R], acc_sh.at[ebuf.at[p % ER, s_row]],
            sem_s).wait()


def _sc1_body(x_hbm, eidx, zeros_h, iota_h,
              acc_out, degA_out,
              ebuf, rowbuf, degA_t, iota_t,
              acc_sh, dA_sh, sem_i, sem_g, sem_s):
    c = lax.axis_index("c")
    s = lax.axis_index("s")
    pltpu.sync_copy(iota_h, iota_t)
    pltpu.sync_copy(zeros_h.at[pl.ds(0, DG)], degA_t)
    base = s * RPT
    pltpu.sync_copy(zeros_h.at[pl.ds(0, RPT)], acc_sh.at[pl.ds(base, RPT)])

    @pl.when(s < DG // 8)
    def _():
        pltpu.sync_copy(zeros_h.at[pl.ds(0, 8)], dA_sh.at[pl.ds(s * 8, 8)])

    plsc.subcore_barrier()

    def hist(b):
        for v in range(K // L):
            dv = ebuf[b, 1, pl.ds(v * L, L)]
            cnt, last = plsc.scan_count(dv)
            plsc.addupdate_scatter(
                degA_t,
                [lax.shift_right_logical(dv, 7), jnp.bitwise_and(dv, 127)],
                cnt.astype(jnp.float32), mask=last)

    _prop_loop(x_hbm, eidx, c, s, ebuf, rowbuf, acc_sh,
               sem_i, sem_g, sem_s, 0, 1, hist)
    pltpu.sync_copy(degA_t, dA_sh.at[iota_t.at[0]], add=True)
    plsc.subcore_barrier()
    pltpu.sync_copy(acc_sh.at[pl.ds(base, RPT)], acc_out.at[c, pl.ds(base, RPT)])

    @pl.when(s < DG // 8)
    def _():
        pltpu.sync_copy(dA_sh.at[pl.ds(s * 8, 8)], degA_out.at[c, pl.ds(s * 8, 8)])


def _sc2_body(s2_hbm, eidx, zeros_h, iota_h,
              acc_out, degR_out,
              ebuf, rowbuf, degR_t, iota_t, acc_sh, dR_sh,
              sem_i, sem_g, sem_s):
    c = lax.axis_index("c")
    s = lax.axis_index("s")
    pltpu.sync_copy(iota_h, iota_t)
    pltpu.sync_copy(zeros_h.at[pl.ds(0, DG)], degR_t)
    base = s * RPT
    pltpu.sync_copy(zeros_h.at[pl.ds(0, RPT)], acc_sh.at[pl.ds(base, RPT)])

    @pl.when(s < DG // 8)
    def _():
        pltpu.sync_copy(zeros_h.at[pl.ds(0, 8)], dR_sh.at[pl.ds(s * 8, 8)])

    plsc.subcore_barrier()

    def hist(b):
        for v in range(K // L):
            sv = ebuf[b, 0, pl.ds(v * L, L)]
            cnt, last = plsc.scan_count(sv)
            plsc.addupdate_scatter(
                degR_t,
                [lax.shift_right_logical(sv, 7), jnp.bitwise_and(sv, 127)],
                cnt.astype(jnp.float32), mask=last)

    _prop_loop(s2_hbm, eidx, c, s, ebuf, rowbuf, acc_sh,
               sem_i, sem_g, sem_s, 1, 0, hist)
    pltpu.sync_copy(degR_t, dR_sh.at[iota_t.at[0]], add=True)
    plsc.subcore_barrier()
    pltpu.sync_copy(acc_sh.at[pl.ds(base, RPT)], acc_out.at[c, pl.ds(base, RPT)])

    @pl.when(s < DG // 8)
    def _():
        pltpu.sync_copy(dR_sh.at[pl.ds(s * 8, 8)], degR_out.at[c, pl.ds(s * 8, 8)])


def _sc_mesh():
    return plsc.VectorSubcoreMesh(
        core_axis_name="c", subcore_axis_name="s", num_cores=NC, num_subcores=NS)


def _sc1(x, eidx, zeros_h, iota_h):
    return pl.kernel(
        _sc1_body,
        out_type=[
            jax.ShapeDtypeStruct((NC, NP, D), jnp.float32),
            jax.ShapeDtypeStruct((NC, DG, 128), jnp.float32),
        ],
        mesh=_sc_mesh(),
        scratch_types=[
            pltpu.VMEM((ER, 2, K), jnp.int32),
            pltpu.VMEM((R, K, D), jnp.float32),
            pltpu.VMEM((DG, 128), jnp.float32),
            pltpu.VMEM((1, DG), jnp.int32),
            pltpu.VMEM_SHARED((NP, D), jnp.float32),
            pltpu.VMEM_SHARED((DG, 128), jnp.float32),
            pltpu.SemaphoreType.DMA,
            pltpu.SemaphoreType.DMA,
            pltpu.SemaphoreType.DMA,
        ],
        compiler_params=pltpu.CompilerParams(needs_layout_passes=False),
        name="gcn_sc_prop1",
    )(x, eidx, zeros_h, iota_h)


def _sc2(s2, eidx, zeros_h, iota_h):
    return pl.kernel(
        _sc2_body,
        out_type=[
            jax.ShapeDtypeStruct((NC, NP, D), jnp.float32),
            jax.ShapeDtypeStruct((NC, DG, 128), jnp.float32),
        ],
        mesh=_sc_mesh(),
        scratch_types=[
            pltpu.VMEM((ER, 2, K), jnp.int32),
            pltpu.VMEM((R, K, D), jnp.float32),
            pltpu.VMEM((DG, 128), jnp.float32),
            pltpu.VMEM((1, DG), jnp.int32),
            pltpu.VMEM_SHARED((NP, D), jnp.float32),
            pltpu.VMEM_SHARED((DG, 128), jnp.float32),
            pltpu.SemaphoreType.DMA,
            pltpu.SemaphoreType.DMA,
            pltpu.SemaphoreType.DMA,
        ],
        compiler_params=pltpu.CompilerParams(needs_layout_passes=False),
        name="gcn_sc_prop2",
    )(s2, eidx, zeros_h, iota_h)


def _softmax2(att_ref):
    ar = att_ref[...]
    m = jnp.maximum(ar[0, 0], ar[0, 1])
    e0 = jnp.exp(ar[0, 0] - m)
    e1 = jnp.exp(ar[0, 1] - m)
    return e0 / (e0 + e1), e1 / (e0 + e1)


def _combine(acc_ref, deg_ref, a0, a1):
    dd = deg_ref[...]
    d0 = dd[0]
    d1 = dd[1]
    inv0 = jnp.where(d0 > 0, a0 / jnp.where(d0 > 0, d0, 1.0), 0.0)
    inv1 = jnp.where(d1 > 0, a1 / jnp.where(d1 > 0, d1, 1.0), 0.0)
    z = acc_ref[0] * inv0 + acc_ref[1] * inv1
    cmask = jnp.where(d0 > 0, a0, 0.0) + jnp.where(d1 > 0, a1, 0.0)
    return z, cmask


def _tc1_body(acc_ref, deg_ref, att_ref, w1_ref, b1_ref, w2_ref, b2_ref, out_ref):
    a0, a1 = _softmax2(att_ref)
    z, cmask = _combine(acc_ref, deg_ref, a0, a1)
    h = jnp.dot(z, w1_ref[...], preferred_element_type=jnp.float32)
    h = h + cmask * b1_ref[...]
    h = jnp.where(h > 0, h, 0.2 * h)
    s2 = jnp.dot(h, w2_ref[...], preferred_element_type=jnp.float32)
    out_ref[...] = s2 + b2_ref[...]


def _tc1(acc, deg, att, W1, b1, W2, b2):
    return pl.pallas_call(
        _tc1_body,
        grid=(N // BN,),
        in_specs=[
            pl.BlockSpec((NC, BN, D), lambda i: (0, i, 0)),
            pl.BlockSpec((NC, BN, 1), lambda i: (0, i, 0)),
            pl.BlockSpec((1, 2), lambda i: (0, 0)),
            pl.BlockSpec((D, DH), lambda i: (0, 0)),
            pl.BlockSpec((1, DH), lambda i: (0, 0)),
            pl.BlockSpec((DH, D), lambda i: (0, 0)),
            pl.BlockSpec((1, D), lambda i: (0, 0)),
        ],
        out_specs=pl.BlockSpec((BN, D), lambda i: (i, 0)),
        out_shape=jax.ShapeDtypeStruct((NP, D), jnp.float32),
        name="gcn_tc_dense",
    )(acc, deg, att, W1, b1, W2, b2)


def _tc2_body(acc_ref, deg_ref, att_ref, out_ref):
    a0, a1 = _softmax2(att_ref)
    o, _ = _combine(acc_ref, deg_ref, a0, a1)
    nrm = jnp.maximum(jnp.sqrt(jnp.sum(o * o, axis=1, keepdims=True)), 1e-12)
    out_ref[...] = o / nrm


def _tc2(acc, deg, att):
    return pl.pallas_call(
        _tc2_body,
        grid=(N // BN,),
        in_specs=[
            pl.BlockSpec((NC, BN, D), lambda i: (0, i, 0)),
            pl.BlockSpec((NC, BN, 1), lambda i: (0, i, 0)),
            pl.BlockSpec((1, 2), lambda i: (0, 0)),
        ],
        out_specs=pl.BlockSpec((BN, D), lambda i: (i, 0)),
        out_shape=jax.ShapeDtypeStruct((N, D), jnp.float32),
        name="gcn_tc_norm",
    )(acc, deg, att)


def _deg_to_col(deg):
    return deg.reshape(NC, DG * 128, 1)


def kernel(x, W1, b1, W2, b2, a_att, r_att, src0, dst0, src1, dst1):
    eidx = jnp.stack([
        jnp.stack([_pad_edges(src0), _pad_edges(dst0)], axis=2),
        jnp.stack([_pad_edges(src1), _pad_edges(dst1)], axis=2),
    ])
    x_pad = jnp.concatenate([x, jnp.zeros((NP - N, D), jnp.float32)])
    zeros_h = jnp.zeros((RPT, D), jnp.float32)
    iota_h = jnp.arange(DG, dtype=jnp.int32).reshape(1, DG)

    accA, degA = _sc1(x_pad, eidx, zeros_h, iota_h)
    s2 = _tc1(accA, _deg_to_col(degA), a_att.reshape(1, 2), W1,
              b1.reshape(1, DH), W2, b2.reshape(1, D))
    accR, degR = _sc2(s2, eidx, zeros_h, iota_h)
    return _tc2(accR, _deg_to_col(degR), r_att.reshape(1, 2))
